# baseline (speedup 1.0000x reference)
import functools

import jax
import jax.numpy as jnp
from jax import lax
from jax.experimental import pallas as pl
from jax.experimental.pallas import tpu as pltpu
from jax.experimental.pallas import tpu_sc as plsc

L = 16
NC, NS = 2, 16
NW = NC * NS
CHUNK = 128
TCW = 128


def _neg_sqrt(x):
    xi = plsc.bitcast(x, jnp.int32)
    y = plsc.bitcast(jnp.int32(0x5F3759DF) - (xi >> 1), jnp.float32)
    for _ in range(3):
        y = y * (1.5 - 0.5 * x * y * y)
    return -(x * y)


def _relayout_body(ent_t_hbm, tail_t_hbm, out_hbm,
                   ibuf, obuf, tibuf, tobuf, insem, outsem,
                   *, emb_dim, n_full, tail):
    wid = lax.axis_index("s") * NC + lax.axis_index("c")
    lanes = lax.iota(jnp.int32, L)
    groups = TCW // L
    count = (n_full - wid + NW - 1) // NW

    def col_of(j):
        return jnp.minimum(wid + NW * j, (count - 1) * NW + wid)

    def fire_in(j, s):
        c0 = col_of(j) * TCW
        return pltpu.async_copy(
            ent_t_hbm.at[pl.ds(0, emb_dim), pl.ds(c0, TCW)], ibuf.at[s],
            insem)

    fire_in(0, 0)
    fire_in(1, 1)

    rv = [m * L + lanes for m in range(groups)]

    def transpose_col(j, s, wait_out):
        sv = jnp.zeros((L,), jnp.int32) + s
        pltpu.make_async_copy(
            ent_t_hbm.at[pl.ds(0, emb_dim), pl.ds(0, TCW)], ibuf.at[s],
            insem).wait()
        if wait_out:
            pltpu.make_async_copy(
                obuf.at[0], out_hbm.at[pl.ds(0, TCW)], outsem).wait()

        def d_body(d, _):
            dv = jnp.zeros((L,), jnp.int32) + d
            for m in range(groups):
                val = ibuf.at[s].at[d][pl.ds(m * L, L)]
                plsc.store_scatter(obuf, [sv, rv[m], dv], val)
            return 0

        lax.fori_loop(0, emb_dim, d_body, 0)
        pltpu.async_copy(obuf.at[s],
                         out_hbm.at[pl.ds(col_of(j) * TCW, TCW)], outsem)
        fire_in(j + 2, s)

    transpose_col(0, 0, False)
    transpose_col(1, 1, False)

    def col_body(j, _):
        transpose_col(j, j & 1, True)
        return 0

    lax.fori_loop(2, count, col_body, 0)

    for s in (0, 1):
        pltpu.make_async_copy(
            ent_t_hbm.at[pl.ds(0, emb_dim), pl.ds(0, TCW)], ibuf.at[s],
            insem).wait()
        pltpu.make_async_copy(
            obuf.at[s], out_hbm.at[pl.ds(0, TCW)], outsem).wait()

    @pl.when(wid == NW - 1)
    def _tail():
        pltpu.sync_copy(tail_t_hbm, tibuf)

        def d_body(d, _):
            dv = jnp.zeros((L,), jnp.int32) + d
            for m in range(tail // L):
                val = tibuf.at[d][pl.ds(m * L, L)]
                plsc.store_scatter(tobuf, [rv[m], dv], val)
            return 0

        lax.fori_loop(0, emb_dim, d_body, 0)
        pltpu.sync_copy(tobuf, out_hbm.at[pl.ds(n_full * TCW, tail)])


def _gather_body(heads_hbm, rels_hbm, tails_hbm, ent_hbm, rel_hbm, out_hbm,
                 hidx_v, ridx_v, tidx_v, h_v, r_v, t_v, out_v,
                 isem, hsem, rsem, tsem, *, rows_per_worker, emb_dim):
    wid = lax.axis_index("s") * NC + lax.axis_index("c")
    base = wid * rows_per_worker
    n_chunks = rows_per_worker // CHUNK
    groups = CHUNK // L

    ic = []
    for k in range(n_chunks):
        off = base + k * CHUNK
        ic.append(pltpu.async_copy(
            heads_hbm.at[pl.ds(off, CHUNK)], hidx_v.at[k], isem))
        ic.append(pltpu.async_copy(
            rels_hbm.at[pl.ds(off, CHUNK)], ridx_v.at[k], isem))
        ic.append(pltpu.async_copy(
            tails_hbm.at[pl.ds(off, CHUNK)], tidx_v.at[k], isem))
    for c in ic:
        c.wait()

    gc = []
    for k in range(n_chunks):
        gc.append(pltpu.async_copy(ent_hbm.at[hidx_v.at[k]], h_v.at[k], hsem))
        gc.append(pltpu.async_copy(rel_hbm.at[ridx_v.at[k]], r_v.at[k], rsem))
        gc.append(pltpu.async_copy(ent_hbm.at[tidx_v.at[k]], t_v.at[k], tsem))

    for k in range(n_chunks):
        gc[3 * k].wait()
        gc[3 * k + 1].wait()
        gc[3 * k + 2].wait()

        def group_body(i, _, k=k):
            rows = i * L + lax.iota(jnp.int32, 16)
            col0 = jnp.zeros((16,), jnp.int32)
            acc = jnp.zeros((16,), jnp.float32)
            for j in range(emb_dim):
                col = col0 + j
                h = plsc.load_gather(h_v.at[k], [rows, col])
                r = plsc.load_gather(r_v.at[k], [rows, col])
                t = plsc.load_gather(t_v.at[k], [rows, col])
                d = h + r - t
                acc = acc + d * d
            out_v[pl.ds(k * CHUNK + i * L, L)] = _neg_sqrt(acc)
            return 0

        lax.fori_loop(0, groups, group_body, 0)

    pltpu.sync_copy(out_v, out_hbm.at[pl.ds(base, rows_per_worker)])


def kernel(heads, rels, tails, ent_embeds, rel_embeds):
    batch = heads.shape[0]
    ent_num, emb_dim = ent_embeds.shape
    rows_per_worker = batch // NW
    n_full = ent_num // TCW
    tail = ent_num - n_full * TCW

    mesh = plsc.VectorSubcoreMesh(core_axis_name="c", subcore_axis_name="s")

    relayout = pl.kernel(
        functools.partial(_relayout_body, emb_dim=emb_dim, n_full=n_full,
                          tail=tail),
        out_type=jax.ShapeDtypeStruct((ent_num, emb_dim), jnp.float32),
        mesh=mesh,
        compiler_params=pltpu.CompilerParams(needs_layout_passes=False,
                                             use_tc_tiling_on_sc=True),
        scratch_types=[
            pltpu.VMEM((2, emb_dim, TCW), jnp.float32),
            pltpu.VMEM((2, TCW, emb_dim), jnp.float32),
            pltpu.VMEM((emb_dim, tail), jnp.float32),
            pltpu.VMEM((tail, emb_dim), jnp.float32),
            pltpu.SemaphoreType.DMA,
            pltpu.SemaphoreType.DMA,
        ],
    )

    n_chunks = rows_per_worker // CHUNK
    gather = pl.kernel(
        functools.partial(_gather_body, rows_per_worker=rows_per_worker,
                          emb_dim=emb_dim),
        out_type=jax.ShapeDtypeStruct((batch,), jnp.float32),
        mesh=mesh,
        compiler_params=pltpu.CompilerParams(needs_layout_passes=False,
                                             use_tc_tiling_on_sc=False),
        scratch_types=[
            pltpu.VMEM((n_chunks, CHUNK), jnp.int32),
            pltpu.VMEM((n_chunks, CHUNK), jnp.int32),
            pltpu.VMEM((n_chunks, CHUNK), jnp.int32),
            pltpu.VMEM((n_chunks, CHUNK, emb_dim), jnp.float32),
            pltpu.VMEM((n_chunks, CHUNK, emb_dim), jnp.float32),
            pltpu.VMEM((n_chunks, CHUNK, emb_dim), jnp.float32),
            pltpu.VMEM((rows_per_worker,), jnp.float32),
            pltpu.SemaphoreType.DMA,
            pltpu.SemaphoreType.DMA,
            pltpu.SemaphoreType.DMA,
            pltpu.SemaphoreType.DMA,
        ],
    )

    ent_t = ent_embeds.T
    tail_t = ent_embeds[n_full * TCW:].T
    ent_rows = relayout(ent_t, tail_t)
    return gather(heads, rels, tails, ent_rows, rel_embeds)
